# baseline (speedup 1.0000x reference)
import functools

import jax
import jax.numpy as jnp
from jax import lax
from jax.experimental import pallas as pl
from jax.experimental.pallas import tpu as pltpu
from jax.experimental.pallas import tpu_sc as plsc

N_TOKEN = 1000000
D_EMBED = 64
EMB_SCALE = D_EMBED ** 0.5

_S = 16384
_T = 50
_NSPLIT = 4
_SS = _S // _NSPLIT
_NW = 32
_S_PER_W = _SS // _NW
_C = 16
_TOK = _C * _T
_NCH = _S_PER_W // _C
_G = 16
_NG = _TOK // _G
_WG = 16
_LANES = 16


def _emb_body(idx_hbm, table_hbm, out_hbm, idx_v, rows_v, sem):
    wid = lax.axis_index("s") * 2 + lax.axis_index("c")
    s0 = wid * _S_PER_W

    def chunk_body(g, _):
        sb = s0 + g * _C
        tb = sb * _T
        pltpu.sync_copy(idx_hbm.at[pl.ds(pl.multiple_of(tb, _TOK), _TOK)], idx_v)

        def fire_group(q, _):
            v = idx_v[pl.ds(q * _G, _G)]
            for k in range(_G):
                pltpu.async_copy(
                    table_hbm.at[pl.ds(v[k], 1)],
                    rows_v.at[pl.ds(q * _G + k, 1)],
                    sem,
                )

            @pl.when(q >= _WG)
            def _():
                pltpu.make_async_copy(
                    table_hbm.at[pl.ds(0, _G)],
                    rows_v.at[pl.ds(0, _G)],
                    sem,
                ).wait()

            return 0

        lax.fori_loop(0, _NG, fire_group, 0)

        pltpu.make_async_copy(
            table_hbm.at[pl.ds(0, _WG * _G)],
            rows_v.at[pl.ds(0, _WG * _G)],
            sem,
        ).wait()

        def scale_row(r, _):
            for k in range(D_EMBED // _LANES):
                sl = pl.ds(k * _LANES, _LANES)
                rows_v[r, sl] = rows_v[r, sl] * EMB_SCALE
            return 0

        lax.fori_loop(0, _TOK, scale_row, 0)

        def write_sample(c, _):
            pltpu.sync_copy(
                rows_v.at[pl.ds(c * _T, _T)],
                out_hbm.at[sb + c],
            )
            return 0

        lax.fori_loop(0, _C, write_sample, 0)
        return 0

    lax.fori_loop(0, _NCH, chunk_body, 0)


_mesh = plsc.VectorSubcoreMesh(core_axis_name="c", subcore_axis_name="s")

_emb_call = functools.partial(
    pl.kernel,
    mesh=_mesh,
    out_type=jax.ShapeDtypeStruct((_SS, _T, D_EMBED), jnp.float32),
    scratch_types=[
        pltpu.VMEM((_TOK,), jnp.int32),
        pltpu.VMEM((_TOK, D_EMBED), jnp.float32),
        pltpu.SemaphoreType.DMA,
    ],
)(_emb_body)


@jax.jit
def kernel(inp, emb_weight):
    idx = inp.reshape(-1).astype(jnp.int32)
    parts = [
        _emb_call(idx[k * _SS * _T:(k + 1) * _SS * _T], emb_weight)
        for k in range(_NSPLIT)
    ]
    return jnp.concatenate(parts, axis=0)


# speedup vs baseline: 1.0721x; 1.0721x over previous
import functools

import jax
import jax.numpy as jnp
from jax import lax
from jax.experimental import pallas as pl
from jax.experimental.pallas import tpu as pltpu
from jax.experimental.pallas import tpu_sc as plsc

N_TOKEN = 1000000
D_EMBED = 64
EMB_SCALE = D_EMBED ** 0.5

_S = 16384
_T = 50
_NW = 32
_S_PER_W = _S // _NW
_C = 8
_TOK = _C * _T
_NCH = _S_PER_W // _C
_G = 16
_NG = _TOK // _G
_WG = 16
_LANES = 16


def _emb_body(idx_hbm, table_hbm, out_hbm, idx0, idx1, rows0, rows1,
              sem_g0, sem_g1, sem_w0, sem_w1):
    wid = lax.axis_index("s") * 2 + lax.axis_index("c")
    s0 = wid * _S_PER_W

    bufs = ((idx0, rows0, sem_g0, sem_w0), (idx1, rows1, sem_g1, sem_w1))

    def stage_idx(g, idx_v):
        tb = (s0 + g * _C) * _T
        pltpu.sync_copy(idx_hbm.at[pl.ds(pl.multiple_of(tb, _TOK), _TOK)], idx_v)

    def gather_chunk(idx_v, rows_v, sem_g):
        def fire_group(q, _):
            v = idx_v[pl.ds(q * _G, _G)]
            for k in range(_G):
                pltpu.async_copy(
                    table_hbm.at[pl.ds(v[k], 1)],
                    rows_v.at[pl.ds(q * _G + k, 1)],
                    sem_g,
                )

            @pl.when(q >= _WG)
            def _():
                pltpu.make_async_copy(
                    table_hbm.at[pl.ds(0, _G)],
                    rows_v.at[pl.ds(0, _G)],
                    sem_g,
                ).wait()

            return 0

        lax.fori_loop(0, _NG, fire_group, 0)
        pltpu.make_async_copy(
            table_hbm.at[pl.ds(0, _WG * _G)],
            rows_v.at[pl.ds(0, _WG * _G)],
            sem_g,
        ).wait()

        def scale_row(r, _):
            for k in range(D_EMBED // _LANES):
                sl = pl.ds(k * _LANES, _LANES)
                rows_v[r, sl] = rows_v[r, sl] * EMB_SCALE
            return 0

        lax.fori_loop(0, _TOK, scale_row, 0)

    def write_chunk(g, rows_v, sem_w):
        sb = s0 + g * _C

        def write_sample(c, _):
            pltpu.async_copy(
                rows_v.at[pl.ds(c * _T, _T)],
                out_hbm.at[sb + c],
                sem_w,
            )
            return 0

        lax.fori_loop(0, _C, write_sample, 0)

    def wait_writes(rows_v, sem_w):
        pltpu.make_async_copy(rows_v, out_hbm.at[pl.ds(0, _C)], sem_w).wait()

    def process(g, idx_v, rows_v, sem_g, sem_w, first):
        stage_idx(g, idx_v)

        @pl.when(jnp.logical_not(first))
        def _():
            wait_writes(rows_v, sem_w)

        gather_chunk(idx_v, rows_v, sem_g)
        write_chunk(g, rows_v, sem_w)

    def pair_body(j, _):
        i0, r0, sg0, sw0 = bufs[0]
        i1, r1, sg1, sw1 = bufs[1]
        process(2 * j, i0, r0, sg0, sw0, j == 0)
        process(2 * j + 1, i1, r1, sg1, sw1, j == 0)
        return 0

    lax.fori_loop(0, _NCH // 2, pair_body, 0)

    wait_writes(bufs[0][1], bufs[0][3])
    wait_writes(bufs[1][1], bufs[1][3])


_mesh = plsc.VectorSubcoreMesh(core_axis_name="c", subcore_axis_name="s")

_emb_call = functools.partial(
    pl.kernel,
    mesh=_mesh,
    out_type=jax.ShapeDtypeStruct((_S, _T, D_EMBED), jnp.float32),
    scratch_types=[
        pltpu.VMEM((_TOK,), jnp.int32),
        pltpu.VMEM((_TOK,), jnp.int32),
        pltpu.VMEM((_TOK, D_EMBED), jnp.float32),
        pltpu.VMEM((_TOK, D_EMBED), jnp.float32),
        pltpu.SemaphoreType.DMA,
        pltpu.SemaphoreType.DMA,
        pltpu.SemaphoreType.DMA,
        pltpu.SemaphoreType.DMA,
    ],
)(_emb_body)


@jax.jit
def kernel(inp, emb_weight):
    idx = inp.reshape(-1).astype(jnp.int32)
    return _emb_call(idx, emb_weight)
